# baseline (speedup 1.0000x reference)
import functools
import math

import jax
import jax.numpy as jnp
from jax.experimental import pallas as pl

D = 1024
E = 8
H = 1024
DOUT = 1024
FFN = 4096
NHEADS = 8
TOPK = 2
HD = DOUT // NHEADS


def _pool_gate_body(x_ref, attn_w_ref, gate_w_ref, xf_ref, comb_ref):
    aw = attn_w_ref[...]
    dn = (((1,), (0,)), ((), ()))
    O = x_ref.shape[1]
    xo = [x_ref[:, o, :] for o in range(O)]
    ls = [jax.lax.dot_general(xi, aw, dn, preferred_element_type=jnp.float32)
          for xi in xo]
    m = ls[0]
    for l in ls[1:]:
        m = jnp.maximum(m, l)
    es = [jnp.exp(l - m) for l in ls]
    ssum = es[0]
    for e_ in es[1:]:
        ssum = ssum + e_
    xa = xo[0] * (es[0] / ssum)
    for o in range(1, O):
        xa = xa + xo[o] * (es[o] / ssum)
    xf_ref[...] = xa.astype(jnp.bfloat16)

    g = jax.lax.dot_general(xa, gate_w_ref[...],
                            (((1,), (0,)), ((), ())),
                            preferred_element_type=jnp.float32)
    tb = g.shape[0]
    iota = jax.lax.broadcasted_iota(jnp.int32, (tb, E), 1)
    m1 = jnp.max(g, axis=1, keepdims=True)
    i1 = jnp.min(jnp.where(g == m1, iota, E), axis=1, keepdims=True)
    mask1 = iota == i1
    neg = jnp.full_like(g, -jnp.inf)
    g2 = jnp.where(mask1, neg, g)
    m2 = jnp.max(g2, axis=1, keepdims=True)
    i2 = jnp.min(jnp.where(g2 == m2, iota, E), axis=1, keepdims=True)
    mask2 = iota == i2
    d = jnp.exp(m2 - m1)
    w1 = 1.0 / (1.0 + d)
    w2 = d * w1
    comb_ref[...] = (mask1.astype(jnp.float32) * w1
                     + mask2.astype(jnp.float32) * w2)


def _pool_gate(x4, attn_w, gate_w):
    T = x4.shape[0]
    O = x4.shape[1]
    TB = 256
    grid = (T // TB,)
    return pl.pallas_call(
        _pool_gate_body,
        grid=grid,
        in_specs=[
            pl.BlockSpec((TB, O, D), lambda i: (i, 0, 0)),
            pl.BlockSpec((D, 1), lambda i: (0, 0)),
            pl.BlockSpec((D, E), lambda i: (0, 0)),
        ],
        out_specs=[
            pl.BlockSpec((TB, D), lambda i: (i, 0)),
            pl.BlockSpec((TB, E), lambda i: (i, 0)),
        ],
        out_shape=[
            jax.ShapeDtypeStruct((T, D), jnp.bfloat16),
            jax.ShapeDtypeStruct((T, E), jnp.float32),
        ],
    )(x4, attn_w, gate_w)


def _moe_body(xf_ref, comb_ref, W1_ref, b1_ref, W2_ref, b2_ref, out_ref):
    e = pl.program_id(0)
    xf = xf_ref[...]
    b1 = b1_ref[pl.ds(e, 1), :]
    b2 = b2_ref[pl.ds(e, 1), :]
    h = jax.lax.dot_general(xf, W1_ref[0], (((1,), (0,)), ((), ())),
                            preferred_element_type=jnp.float32) + b1
    h = jnp.maximum(h, 0.0).astype(jnp.bfloat16)
    y = jax.lax.dot_general(h, W2_ref[0], (((1,), (0,)), ((), ())),
                            preferred_element_type=jnp.float32) + b2
    comb = comb_ref[...]
    sel = (jax.lax.broadcasted_iota(jnp.int32, comb.shape, 1) == e)
    c = jnp.sum(jnp.where(sel, comb, 0.0), axis=1, keepdims=True)
    contrib = c * y

    @pl.when(e == 0)
    def _():
        out_ref[...] = contrib

    @pl.when(e != 0)
    def _():
        out_ref[...] = out_ref[...] + contrib


def _moe(xf, comb, W1, b1, W2, b2):
    T = xf.shape[0]
    return pl.pallas_call(
        _moe_body,
        grid=(E,),
        in_specs=[
            pl.BlockSpec((T, D), lambda e: (0, 0)),
            pl.BlockSpec((T, E), lambda e: (0, 0)),
            pl.BlockSpec((1, D, H), lambda e: (e, 0, 0)),
            pl.BlockSpec((E, H), lambda e: (0, 0)),
            pl.BlockSpec((1, H, DOUT), lambda e: (e, 0, 0)),
            pl.BlockSpec((E, DOUT), lambda e: (0, 0)),
        ],
        out_specs=pl.BlockSpec((T, DOUT), lambda e: (0, 0)),
        out_shape=jax.ShapeDtypeStruct((T, DOUT), jnp.float32),
    )(xf, comb, W1, b1, W2, b2)


def _attn_body(ef_ref, wq_ref, bq_ref, wk_ref, bk_ref, wv_ref, bv_ref,
               wo_ref, bo_ref, out_ref):
    ef = ef_ref[0].astype(jnp.bfloat16)
    dn = (((1,), (0,)), ((), ()))
    q = (jax.lax.dot_general(ef, wq_ref[...], dn,
                             preferred_element_type=jnp.float32) + bq_ref[...]).astype(jnp.bfloat16)
    k = (jax.lax.dot_general(ef, wk_ref[...], dn,
                             preferred_element_type=jnp.float32) + bk_ref[...]).astype(jnp.bfloat16)
    v = (jax.lax.dot_general(ef, wv_ref[...], dn,
                             preferred_element_type=jnp.float32) + bv_ref[...]).astype(jnp.bfloat16)
    scale = 1.0 / math.sqrt(HD)
    outs = []
    for hh in range(NHEADS):
        sl = slice(hh * HD, (hh + 1) * HD)
        qh = q[:, sl]
        kh = k[:, sl]
        vh = v[:, sl]
        s = jax.lax.dot_general(qh, kh, (((1,), (1,)), ((), ())),
                                preferred_element_type=jnp.float32) * scale
        m = jnp.max(s, axis=1, keepdims=True)
        p = jnp.exp(s - m)
        p = (p / jnp.sum(p, axis=1, keepdims=True)).astype(jnp.bfloat16)
        outs.append(jax.lax.dot_general(p, vh, dn,
                                        preferred_element_type=jnp.float32))
    o = jnp.concatenate(outs, axis=1).astype(jnp.bfloat16)
    out_ref[0] = jax.lax.dot_general(o, wo_ref[...], dn,
                                     preferred_element_type=jnp.float32) + bo_ref[...]


def _attn(ef, wq, bq, wk, bk, wv, bv, wo, bo):
    B, N, _ = ef.shape
    wspec = pl.BlockSpec((DOUT, DOUT), lambda b: (0, 0))
    bspec = pl.BlockSpec((1, DOUT), lambda b: (0, 0))
    return pl.pallas_call(
        _attn_body,
        grid=(B,),
        in_specs=[
            pl.BlockSpec((1, N, DOUT), lambda b: (b, 0, 0)),
            wspec, bspec, wspec, bspec, wspec, bspec, wspec, bspec,
        ],
        out_specs=pl.BlockSpec((1, N, DOUT), lambda b: (b, 0, 0)),
        out_shape=jax.ShapeDtypeStruct((B, N, DOUT), jnp.float32),
    )(ef, wq, bq.reshape(1, DOUT), wk, bk.reshape(1, DOUT),
      wv, bv.reshape(1, DOUT), wo, bo.reshape(1, DOUT))


def _ffn_body(x_ref, f1w_ref, f1b_ref, f2w_ref, f2b_ref, out_ref):
    dn = (((1,), (0,)), ((), ()))
    h = jax.lax.dot_general(x_ref[...].astype(jnp.bfloat16), f1w_ref[...], dn,
                            preferred_element_type=jnp.float32) + f1b_ref[...]
    h = jnp.maximum(h, 0.0).astype(jnp.bfloat16)
    out_ref[...] = jax.lax.dot_general(h, f2w_ref[...], dn,
                                       preferred_element_type=jnp.float32) + f2b_ref[...]


def _ffn(x2, f1w, f1b, f2w, f2b):
    T = x2.shape[0]
    TB = 256
    return pl.pallas_call(
        _ffn_body,
        grid=(T // TB,),
        in_specs=[
            pl.BlockSpec((TB, DOUT), lambda i: (i, 0)),
            pl.BlockSpec((DOUT, FFN), lambda i: (0, 0)),
            pl.BlockSpec((1, FFN), lambda i: (0, 0)),
            pl.BlockSpec((FFN, DOUT), lambda i: (0, 0)),
            pl.BlockSpec((1, DOUT), lambda i: (0, 0)),
        ],
        out_specs=pl.BlockSpec((TB, DOUT), lambda i: (i, 0)),
        out_shape=jax.ShapeDtypeStruct((T, DOUT), jnp.float32),
    )(x2, f1w, f1b.reshape(1, FFN), f2w, f2b.reshape(1, DOUT))


@jax.jit
def kernel(x, attn_w, gate_w, W1, b1, W2, b2, wq, bq, wk, bk, wv, bv,
           wo, bo, f1w, f1b, f2w, f2b):
    B, N, O, d = x.shape
    T = B * N
    x4 = x.reshape(T, O, d)
    bf = jnp.bfloat16
    xf, comb = _pool_gate(x4, attn_w, gate_w)
    moe = _moe(xf, comb, W1.astype(bf), b1, W2.astype(bf), b2)
    ef = moe.reshape(B, N, DOUT)
    rel = _attn(ef, wq.astype(bf), bq, wk.astype(bf), bk,
                wv.astype(bf), bv, wo.astype(bf), bo)
    out = _ffn(rel.reshape(T, DOUT), f1w.astype(bf), f1b, f2w.astype(bf), f2b)
    return out.reshape(B, N, DOUT)


# speedup vs baseline: 1.3301x; 1.3301x over previous
import math

import jax
import jax.numpy as jnp
from jax.experimental import pallas as pl
from jax.experimental.pallas import tpu as pltpu

D = 1024
E = 8
H = 1024
DOUT = 1024
FFN = 4096
NHEADS = 8
HD = DOUT // NHEADS
T = 1024
N = 512
B = 2
XCH = 4
XTB = T // XCH
NSLOT = 4


def _mega_body(x_hbm, attn_w_ref, gate_w_ref, W1_hbm, b1_ref, W2_hbm, b2_ref,
               wq_hbm, bq_ref, wk_hbm, bk_ref, wv_hbm, bv_ref, wo_hbm, bo_ref,
               f1w_hbm, f1b_ref, f2w_hbm, f2b_ref,
               out_ref,
               xbuf, xf, comb, ef, q, k, v, wring,
               sem_x, sem_w):
    dn = (((1,), (0,)), ((), ()))
    dnt = (((1,), (1,)), ((), ()))
    f32 = jnp.float32

    def xcopy(t, slot):
        return pltpu.make_async_copy(
            x_hbm.at[pl.ds(t * XTB, XTB), :, :], xbuf.at[slot], sem_x.at[slot])

    def wcopy_w1(e, slot):
        return pltpu.make_async_copy(W1_hbm.at[e], wring.at[slot], sem_w.at[slot])

    def wcopy_w2(e, slot):
        return pltpu.make_async_copy(W2_hbm.at[e], wring.at[slot], sem_w.at[slot])

    def wcopy_plain(src, slot):
        return pltpu.make_async_copy(src, wring.at[slot], sem_w.at[slot])

    def wcopy_f1(j, slot):
        return pltpu.make_async_copy(
            f1w_hbm.at[:, pl.ds(j * 1024, 1024)], wring.at[slot], sem_w.at[slot])

    def wcopy_f2(j, slot):
        return pltpu.make_async_copy(
            f2w_hbm.at[pl.ds(j * 1024, 1024), :], wring.at[slot], sem_w.at[slot])

    xcopy(0, 0).start()
    xcopy(1, 1).start()
    wcopy_w1(0, 0).start()
    wcopy_w2(0, 1).start()
    wcopy_w1(1, 2).start()

    aw = attn_w_ref[...]
    gw = gate_w_ref[...]
    for t in range(XCH):
        xcopy(t, t % 2).wait()
        xc = xbuf[t % 2]
        xo = [xc[:, o, :] for o in range(4)]
        ls = [jax.lax.dot_general(xi, aw, dn, preferred_element_type=f32)
              for xi in xo]
        m = jnp.maximum(jnp.maximum(ls[0], ls[1]), jnp.maximum(ls[2], ls[3]))
        es = [jnp.exp(l - m) for l in ls]
        ssum = (es[0] + es[1]) + (es[2] + es[3])
        xa = xo[0] * (es[0] / ssum)
        for o in range(1, 4):
            xa = xa + xo[o] * (es[o] / ssum)
        xf[pl.ds(t * XTB, XTB), :] = xa

        g = jax.lax.dot_general(xa, gw, dn, preferred_element_type=f32)
        iota = jax.lax.broadcasted_iota(jnp.int32, (XTB, E), 1)
        m1 = jnp.max(g, axis=1, keepdims=True)
        i1 = jnp.min(jnp.where(g == m1, iota, E), axis=1, keepdims=True)
        mask1 = iota == i1
        g2 = jnp.where(mask1, jnp.full_like(g, -jnp.inf), g)
        m2 = jnp.max(g2, axis=1, keepdims=True)
        i2 = jnp.min(jnp.where(g2 == m2, iota, E), axis=1, keepdims=True)
        mask2 = iota == i2
        dd = jnp.exp(m2 - m1)
        w1 = 1.0 / (1.0 + dd)
        w2 = dd * w1
        comb[pl.ds(t * XTB, XTB), :] = (mask1.astype(f32) * w1
                                        + mask2.astype(f32) * w2)
        if t + 2 < XCH:
            xcopy(t + 2, t % 2).start()

    def moe_step(e, _):
        s1 = jax.lax.rem(2 * e, NSLOT)
        s2 = jax.lax.rem(2 * e + 1, NSLOT)
        wcopy_w1(0, s1).wait()
        wcopy_w2(0, s2).wait()
        xfv = xf[...]
        h = jax.lax.dot_general(xfv, wring[s1], dn, preferred_element_type=f32)
        h = jnp.maximum(h + b1_ref[pl.ds(e, 1), :], 0.0)
        y = jax.lax.dot_general(h, wring[s2], dn, preferred_element_type=f32)
        y = y + b2_ref[pl.ds(e, 1), :]
        cmb = comb[...]
        sel = (jax.lax.broadcasted_iota(jnp.int32, cmb.shape, 1) == e)
        c = jnp.sum(jnp.where(sel, cmb, 0.0), axis=1, keepdims=True)
        contrib = c * y

        @pl.when(e == 0)
        def _():
            ef[...] = contrib

        @pl.when(e != 0)
        def _():
            ef[...] = ef[...] + contrib

        @pl.when(e <= 6)
        def _():
            wcopy_w2(jnp.minimum(e + 1, E - 1), jax.lax.rem(2 * e + 3, NSLOT)).start()

        @pl.when(e <= 5)
        def _():
            wcopy_w1(jnp.minimum(e + 2, E - 1), jax.lax.rem(2 * e + 4, NSLOT)).start()

        @pl.when(e == 6)
        def _():
            wcopy_plain(wq_hbm, 0).start()

        @pl.when(e == 7)
        def _():
            wcopy_plain(wk_hbm, 1).start()
            wcopy_plain(wv_hbm, 2).start()
        return 0

    jax.lax.fori_loop(0, E, moe_step, 0)
    wcopy_plain(wo_hbm, 3).start()

    efv = ef[...]
    wcopy_plain(wq_hbm, 0).wait()
    q[...] = jax.lax.dot_general(efv, wring[0], dn,
                                 preferred_element_type=f32) + bq_ref[...]
    wcopy_f1(0, 0).start()
    wcopy_plain(wk_hbm, 1).wait()
    k[...] = jax.lax.dot_general(efv, wring[1], dn,
                                 preferred_element_type=f32) + bk_ref[...]
    wcopy_f2(0, 1).start()
    wcopy_plain(wv_hbm, 2).wait()
    v[...] = jax.lax.dot_general(efv, wring[2], dn,
                                 preferred_element_type=f32) + bv_ref[...]
    wcopy_f1(1, 2).start()

    scale = 1.0 / math.sqrt(HD)

    def head_step(i, _):
        b = i // NHEADS
        hh = jax.lax.rem(i, NHEADS)
        rs = pl.ds(b * N, N)
        cs = pl.ds(hh * HD, HD)
        qh = q[rs, cs]
        kh = k[rs, cs]
        vh = v[rs, cs]
        s = jax.lax.dot_general(qh, kh, dnt, preferred_element_type=f32) * scale
        mm = jnp.max(s, axis=1, keepdims=True)
        p = jnp.exp(s - mm)
        p = p / jnp.sum(p, axis=1, keepdims=True)
        ef[rs, cs] = jax.lax.dot_general(p, vh, dn, preferred_element_type=f32)
        return 0

    jax.lax.fori_loop(0, B * NHEADS, head_step, 0)

    ov = ef[...]
    wcopy_plain(wo_hbm, 3).wait()
    rel = jax.lax.dot_general(ov, wring[3], dn,
                              preferred_element_type=f32) + bo_ref[...]
    xf[...] = rel
    wcopy_f2(1, 3).start()

    relv = xf[...]
    for j in range(4):
        s1 = (2 * j) % NSLOT
        s2 = (2 * j + 1) % NSLOT
        wcopy_f1(j, s1).wait()
        hj = jax.lax.dot_general(relv, wring[s1], dn, preferred_element_type=f32)
        hj = jnp.maximum(hj + f1b_ref[:, pl.ds(j * 1024, 1024)], 0.0)
        if j + 2 < 4:
            wcopy_f1(j + 2, (2 * j + 4) % NSLOT).start()
        wcopy_f2(j, s2).wait()
        yj = jax.lax.dot_general(hj, wring[s2], dn, preferred_element_type=f32)
        if j == 0:
            out_ref[...] = yj
        else:
            out_ref[...] = out_ref[...] + yj
        if j + 2 < 4:
            wcopy_f2(j + 2, (2 * j + 5) % NSLOT).start()
    out_ref[...] = out_ref[...] + f2b_ref[...]


@jax.jit
def kernel(x, attn_w, gate_w, W1, b1, W2, b2, wq, bq, wk, bk, wv, bv,
           wo, bo, f1w, f1b, f2w, f2b):
    x4 = x.reshape(T, 4, D)
    vspec = lambda shape: pl.BlockSpec(shape, lambda: tuple(0 for _ in shape))
    any_spec = pl.BlockSpec(memory_space=pl.ANY)
    out = pl.pallas_call(
        _mega_body,
        grid=(),
        compiler_params=pltpu.CompilerParams(vmem_limit_bytes=67108864),
        in_specs=[
            any_spec,
            vspec((D, 1)),
            vspec((D, E)),
            any_spec,
            vspec((E, H)),
            any_spec,
            vspec((E, DOUT)),
            any_spec,
            vspec((1, DOUT)),
            any_spec,
            vspec((1, DOUT)),
            any_spec,
            vspec((1, DOUT)),
            any_spec,
            vspec((1, DOUT)),
            any_spec,
            vspec((1, FFN)),
            any_spec,
            vspec((1, DOUT)),
        ],
        out_specs=pl.BlockSpec((T, DOUT), lambda: (0, 0)),
        out_shape=jax.ShapeDtypeStruct((T, DOUT), jnp.float32),
        scratch_shapes=[
            pltpu.VMEM((2, XTB, 4, D), jnp.float32),
            pltpu.VMEM((T, D), jnp.float32),
            pltpu.VMEM((T, E), jnp.float32),
            pltpu.VMEM((T, DOUT), jnp.float32),
            pltpu.VMEM((T, DOUT), jnp.float32),
            pltpu.VMEM((T, DOUT), jnp.float32),
            pltpu.VMEM((T, DOUT), jnp.float32),
            pltpu.VMEM((NSLOT, 1024, 1024), jnp.float32),
            pltpu.SemaphoreType.DMA((2,)),
            pltpu.SemaphoreType.DMA((NSLOT,)),
        ],
    )(x4, attn_w, gate_w, W1, b1, W2, b2,
      wq, bq.reshape(1, DOUT), wk, bk.reshape(1, DOUT),
      wv, bv.reshape(1, DOUT), wo, bo.reshape(1, DOUT),
      f1w, f1b.reshape(1, FFN), f2w, f2b.reshape(1, DOUT))
    return out.reshape(B, N, DOUT)


# speedup vs baseline: 1.4878x; 1.1186x over previous
import math

import jax
import jax.numpy as jnp
from jax.experimental import pallas as pl
from jax.experimental.pallas import tpu as pltpu

D = 1024
E = 8
H = 1024
DOUT = 1024
FFN = 4096
NHEADS = 8
HD = DOUT // NHEADS
T = 1024
N = 512
B = 2
XCH = 4
XTB = T // XCH
NSLOT = 4


def _mega_body(x_hbm, attn_w_ref, gate_w_ref, W1_hbm, b1_ref, W2_hbm, b2_ref,
               wq_hbm, bq_ref, wk_hbm, bk_ref, wv_hbm, bv_ref, wo_hbm, bo_ref,
               f1w_hbm, f1b_ref, f2w_hbm, f2b_ref,
               out_ref,
               xbuf, xf, comb, ef, q, k, v, wring,
               sem_x, sem_w):
    dn = (((1,), (0,)), ((), ()))
    dnt = (((1,), (1,)), ((), ()))
    f32 = jnp.float32

    def xcopy(t, slot):
        return pltpu.make_async_copy(
            x_hbm.at[pl.ds(t * XTB, XTB), :, :], xbuf.at[slot], sem_x.at[slot])

    def wcopy_w1(e, slot):
        return pltpu.make_async_copy(W1_hbm.at[e], wring.at[slot], sem_w.at[slot])

    def wcopy_w2(e, slot):
        return pltpu.make_async_copy(W2_hbm.at[e], wring.at[slot], sem_w.at[slot])

    def wcopy_plain(src, slot):
        return pltpu.make_async_copy(src, wring.at[slot], sem_w.at[slot])

    def wcopy_f1(j, slot):
        return pltpu.make_async_copy(
            f1w_hbm.at[:, pl.ds(j * 1024, 1024)], wring.at[slot], sem_w.at[slot])

    def wcopy_f2(j, slot):
        return pltpu.make_async_copy(
            f2w_hbm.at[pl.ds(j * 1024, 1024), :], wring.at[slot], sem_w.at[slot])

    xcopy(0, 0).start()
    xcopy(1, 1).start()
    wcopy_w1(0, 0).start()
    wcopy_w2(0, 1).start()
    wcopy_w1(1, 2).start()

    aw = attn_w_ref[...]
    gw = gate_w_ref[...]
    for t in range(XCH):
        xcopy(t, t % 2).wait()
        xc = xbuf[t % 2]
        xo = [xc[:, o, :] for o in range(4)]
        ls = [jax.lax.dot_general(xi, aw, dn, preferred_element_type=f32)
              for xi in xo]
        m = jnp.maximum(jnp.maximum(ls[0], ls[1]), jnp.maximum(ls[2], ls[3]))
        es = [jnp.exp(l - m) for l in ls]
        ssum = (es[0] + es[1]) + (es[2] + es[3])
        xa = xo[0] * (es[0] / ssum)
        for o in range(1, 4):
            xa = xa + xo[o] * (es[o] / ssum)
        xf[pl.ds(t * XTB, XTB), :] = xa

        g = jax.lax.dot_general(xa, gw, dn, preferred_element_type=f32)
        iota = jax.lax.broadcasted_iota(jnp.int32, (XTB, E), 1)
        m1 = jnp.max(g, axis=1, keepdims=True)
        i1 = jnp.min(jnp.where(g == m1, iota, E), axis=1, keepdims=True)
        mask1 = iota == i1
        g2 = jnp.where(mask1, jnp.full_like(g, -jnp.inf), g)
        m2 = jnp.max(g2, axis=1, keepdims=True)
        i2 = jnp.min(jnp.where(g2 == m2, iota, E), axis=1, keepdims=True)
        mask2 = iota == i2
        dd = jnp.exp(m2 - m1)
        w1 = 1.0 / (1.0 + dd)
        w2 = dd * w1
        comb[pl.ds(t * XTB, XTB), :] = (mask1.astype(f32) * w1
                                        + mask2.astype(f32) * w2)
        if t + 2 < XCH:
            xcopy(t + 2, t % 2).start()

    for e in range(E):
        s1 = (2 * e) % NSLOT
        s2 = (2 * e + 1) % NSLOT
        wcopy_w1(0, s1).wait()
        wcopy_w2(0, s2).wait()
        xfv = xf[...]
        h = jax.lax.dot_general(xfv, wring[s1], dn, preferred_element_type=f32)
        h = jnp.maximum(h + b1_ref[e:e + 1, :], 0.0)
        y = jax.lax.dot_general(h, wring[s2], dn, preferred_element_type=f32)
        y = y + b2_ref[e:e + 1, :]
        cmb = comb[...]
        sel = (jax.lax.broadcasted_iota(jnp.int32, cmb.shape, 1) == e)
        c = jnp.sum(jnp.where(sel, cmb, 0.0), axis=1, keepdims=True)
        contrib = c * y
        if e == 0:
            ef[...] = contrib
        else:
            ef[...] = ef[...] + contrib
        if e <= 6:
            wcopy_w2(e + 1, (2 * e + 3) % NSLOT).start()
        if e <= 5:
            wcopy_w1(e + 2, (2 * e + 4) % NSLOT).start()
        if e == 6:
            wcopy_plain(wq_hbm, 0).start()
        if e == 7:
            wcopy_plain(wk_hbm, 1).start()
            wcopy_plain(wv_hbm, 2).start()
    wcopy_plain(wo_hbm, 3).start()

    efv = ef[...]
    wcopy_plain(wq_hbm, 0).wait()
    q[...] = jax.lax.dot_general(efv, wring[0], dn,
                                 preferred_element_type=f32) + bq_ref[...]
    wcopy_f1(0, 0).start()
    wcopy_plain(wk_hbm, 1).wait()
    k[...] = jax.lax.dot_general(efv, wring[1], dn,
                                 preferred_element_type=f32) + bk_ref[...]
    wcopy_f2(0, 1).start()
    wcopy_plain(wv_hbm, 2).wait()
    v[...] = jax.lax.dot_general(efv, wring[2], dn,
                                 preferred_element_type=f32) + bv_ref[...]
    wcopy_f1(1, 2).start()

    scale = 1.0 / math.sqrt(HD)

    for i in range(B * NHEADS):
        b = i // NHEADS
        hh = i % NHEADS
        rs = slice(b * N, (b + 1) * N)
        cs = slice(hh * HD, (hh + 1) * HD)
        qh = q[rs, cs]
        kh = k[rs, cs]
        vh = v[rs, cs]
        s = jax.lax.dot_general(qh, kh, dnt, preferred_element_type=f32) * scale
        mm = jnp.max(s, axis=1, keepdims=True)
        p = jnp.exp(s - mm)
        p = p / jnp.sum(p, axis=1, keepdims=True)
        ef[rs, cs] = jax.lax.dot_general(p, vh, dn, preferred_element_type=f32)

    ov = ef[...]
    wcopy_plain(wo_hbm, 3).wait()
    rel = jax.lax.dot_general(ov, wring[3], dn,
                              preferred_element_type=f32) + bo_ref[...]
    xf[...] = rel
    wcopy_f2(1, 3).start()

    relv = xf[...]
    for j in range(4):
        s1 = (2 * j) % NSLOT
        s2 = (2 * j + 1) % NSLOT
        wcopy_f1(j, s1).wait()
        hj = jax.lax.dot_general(relv, wring[s1], dn, preferred_element_type=f32)
        hj = jnp.maximum(hj + f1b_ref[:, pl.ds(j * 1024, 1024)], 0.0)
        if j + 2 < 4:
            wcopy_f1(j + 2, (2 * j + 4) % NSLOT).start()
        wcopy_f2(j, s2).wait()
        yj = jax.lax.dot_general(hj, wring[s2], dn, preferred_element_type=f32)
        if j == 0:
            out_ref[...] = yj
        else:
            out_ref[...] = out_ref[...] + yj
        if j + 2 < 4:
            wcopy_f2(j + 2, (2 * j + 5) % NSLOT).start()
    out_ref[...] = out_ref[...] + f2b_ref[...]


@jax.jit
def kernel(x, attn_w, gate_w, W1, b1, W2, b2, wq, bq, wk, bk, wv, bv,
           wo, bo, f1w, f1b, f2w, f2b):
    x4 = x.reshape(T, 4, D)
    vspec = lambda shape: pl.BlockSpec(shape, lambda: tuple(0 for _ in shape))
    any_spec = pl.BlockSpec(memory_space=pl.ANY)
    out = pl.pallas_call(
        _mega_body,
        grid=(),
        compiler_params=pltpu.CompilerParams(vmem_limit_bytes=67108864),
        in_specs=[
            any_spec,
            vspec((D, 1)),
            vspec((D, E)),
            any_spec,
            vspec((E, H)),
            any_spec,
            vspec((E, DOUT)),
            any_spec,
            vspec((1, DOUT)),
            any_spec,
            vspec((1, DOUT)),
            any_spec,
            vspec((1, DOUT)),
            any_spec,
            vspec((1, DOUT)),
            any_spec,
            vspec((1, FFN)),
            any_spec,
            vspec((1, DOUT)),
        ],
        out_specs=pl.BlockSpec((T, DOUT), lambda: (0, 0)),
        out_shape=jax.ShapeDtypeStruct((T, DOUT), jnp.float32),
        scratch_shapes=[
            pltpu.VMEM((2, XTB, 4, D), jnp.float32),
            pltpu.VMEM((T, D), jnp.float32),
            pltpu.VMEM((T, E), jnp.float32),
            pltpu.VMEM((T, DOUT), jnp.float32),
            pltpu.VMEM((T, DOUT), jnp.float32),
            pltpu.VMEM((T, DOUT), jnp.float32),
            pltpu.VMEM((T, DOUT), jnp.float32),
            pltpu.VMEM((NSLOT, 1024, 1024), jnp.float32),
            pltpu.SemaphoreType.DMA((2,)),
            pltpu.SemaphoreType.DMA((NSLOT,)),
        ],
    )(x4, attn_w, gate_w, W1, b1, W2, b2,
      wq, bq.reshape(1, DOUT), wk, bk.reshape(1, DOUT),
      wv, bv.reshape(1, DOUT), wo, bo.reshape(1, DOUT),
      f1w, f1b.reshape(1, FFN), f2w, f2b.reshape(1, DOUT))
    return out.reshape(B, N, DOUT)


# speedup vs baseline: 1.5914x; 1.0696x over previous
import math

import jax
import jax.numpy as jnp
from jax.experimental import pallas as pl
from jax.experimental.pallas import tpu as pltpu

D = 1024
E = 8
H = 1024
DOUT = 1024
FFN = 4096
NHEADS = 8
HD = DOUT // NHEADS
T = 1024
N = 512
B = 2
XCH = 8
XTB = T // XCH
NSLOT = 5
NCHUNK = 28


def _mega_body(x_hbm, attn_w_ref, gate_w_ref, W1_hbm, b1_ref, W2_hbm, b2_ref,
               wq_hbm, bq_ref, wk_hbm, bk_ref, wv_hbm, bv_ref, wo_hbm, bo_ref,
               f1w_hbm, f1b_ref, f2w_hbm, f2b_ref,
               out_ref,
               xbuf, xf, comb, ef, q, k, v, wring,
               sem_x, sem_w):
    dn = (((1,), (0,)), ((), ()))
    dnt = (((1,), (1,)), ((), ()))
    f32 = jnp.float32

    def xcopy(t):
        return pltpu.make_async_copy(
            x_hbm.at[pl.ds(t * XTB, XTB), :, :], xbuf.at[t % 2],
            sem_x.at[t % 2])

    def wchunk(ci):
        slot = ci % NSLOT
        if ci < 16:
            e2, r = divmod(ci, 2)
            src = W1_hbm.at[e2] if r == 0 else W2_hbm.at[e2]
        elif ci < 20:
            src = [wq_hbm, wk_hbm, wv_hbm, wo_hbm][ci - 16]
        else:
            j2, r = divmod(ci - 20, 2)
            if r == 0:
                src = f1w_hbm.at[:, pl.ds(j2 * 1024, 1024)]
            else:
                src = f2w_hbm.at[pl.ds(j2 * 1024, 1024), :]
        return pltpu.make_async_copy(src, wring.at[slot], sem_w.at[slot])

    def issue(ci):
        if ci < NCHUNK:
            wchunk(ci).start()

    xcopy(0).start()
    xcopy(1).start()
    for ci in range(NSLOT):
        issue(ci)

    aw = attn_w_ref[...]
    gw = gate_w_ref[...]
    for t in range(XCH):
        xcopy(t).wait()
        xc = xbuf[t % 2]
        xo = [xc[:, o, :] for o in range(4)]
        ls = [jax.lax.dot_general(xi, aw, dn, preferred_element_type=f32)
              for xi in xo]
        m = jnp.maximum(jnp.maximum(ls[0], ls[1]), jnp.maximum(ls[2], ls[3]))
        es = [jnp.exp(l - m) for l in ls]
        ssum = (es[0] + es[1]) + (es[2] + es[3])
        xa = xo[0] * (es[0] / ssum)
        for o in range(1, 4):
            xa = xa + xo[o] * (es[o] / ssum)
        xf[pl.ds(t * XTB, XTB), :] = xa

        g = jax.lax.dot_general(xa, gw, dn, preferred_element_type=f32)
        iota = jax.lax.broadcasted_iota(jnp.int32, (XTB, E), 1)
        m1 = jnp.max(g, axis=1, keepdims=True)
        i1 = jnp.min(jnp.where(g == m1, iota, E), axis=1, keepdims=True)
        mask1 = iota == i1
        g2 = jnp.where(mask1, jnp.full_like(g, -jnp.inf), g)
        m2 = jnp.max(g2, axis=1, keepdims=True)
        i2 = jnp.min(jnp.where(g2 == m2, iota, E), axis=1, keepdims=True)
        mask2 = iota == i2
        dd = jnp.exp(m2 - m1)
        w1 = 1.0 / (1.0 + dd)
        w2 = dd * w1
        comb[pl.ds(t * XTB, XTB), :] = (mask1.astype(f32) * w1
                                        + mask2.astype(f32) * w2)
        if t + 2 < XCH:
            xcopy(t + 2).start()

    for e in range(E):
        c1, c2 = 2 * e, 2 * e + 1
        wchunk(c1).wait()
        wchunk(c2).wait()
        xfv = xf[...]
        h = jax.lax.dot_general(xfv, wring[c1 % NSLOT], dn,
                                preferred_element_type=f32)
        h = jnp.maximum(h + b1_ref[e:e + 1, :], 0.0)
        y = jax.lax.dot_general(h, wring[c2 % NSLOT], dn,
                                preferred_element_type=f32)
        y = y + b2_ref[e:e + 1, :]
        cmb = comb[...]
        sel = (jax.lax.broadcasted_iota(jnp.int32, cmb.shape, 1) == e)
        c = jnp.sum(jnp.where(sel, cmb, 0.0), axis=1, keepdims=True)
        contrib = c * y
        if e == 0:
            ef[...] = contrib
        else:
            ef[...] = ef[...] + contrib
        issue(c1 + NSLOT)
        issue(c2 + NSLOT)

    efv = ef[...]
    wchunk(16).wait()
    q[...] = jax.lax.dot_general(efv, wring[16 % NSLOT], dn,
                                 preferred_element_type=f32) + bq_ref[...]
    issue(16 + NSLOT)
    wchunk(17).wait()
    k[...] = jax.lax.dot_general(efv, wring[17 % NSLOT], dn,
                                 preferred_element_type=f32) + bk_ref[...]
    issue(17 + NSLOT)
    wchunk(18).wait()
    v[...] = jax.lax.dot_general(efv, wring[18 % NSLOT], dn,
                                 preferred_element_type=f32) + bv_ref[...]
    issue(18 + NSLOT)

    scale = 1.0 / math.sqrt(HD)
    for i in range(B * NHEADS):
        b = i // NHEADS
        hh = i % NHEADS
        rs = slice(b * N, (b + 1) * N)
        cs = slice(hh * HD, (hh + 1) * HD)
        qh = q[rs, cs]
        kh = k[rs, cs]
        vh = v[rs, cs]
        s = jax.lax.dot_general(qh, kh, dnt, preferred_element_type=f32) * scale
        mm = jnp.max(s, axis=1, keepdims=True)
        p = jnp.exp(s - mm)
        p = p / jnp.sum(p, axis=1, keepdims=True)
        ef[rs, cs] = jax.lax.dot_general(p, vh, dn, preferred_element_type=f32)

    ov = ef[...]
    wchunk(19).wait()
    rel = jax.lax.dot_general(ov, wring[19 % NSLOT], dn,
                              preferred_element_type=f32) + bo_ref[...]
    xf[...] = rel
    issue(19 + NSLOT)

    relv = xf[...]
    for j in range(4):
        c1, c2 = 20 + 2 * j, 21 + 2 * j
        wchunk(c1).wait()
        hj = jax.lax.dot_general(relv, wring[c1 % NSLOT], dn,
                                 preferred_element_type=f32)
        hj = jnp.maximum(hj + f1b_ref[:, pl.ds(j * 1024, 1024)], 0.0)
        issue(c1 + NSLOT)
        wchunk(c2).wait()
        yj = jax.lax.dot_general(hj, wring[c2 % NSLOT], dn,
                                 preferred_element_type=f32)
        if j == 0:
            out_ref[...] = yj
        else:
            out_ref[...] = out_ref[...] + yj
        issue(c2 + NSLOT)
    out_ref[...] = out_ref[...] + f2b_ref[...]


@jax.jit
def kernel(x, attn_w, gate_w, W1, b1, W2, b2, wq, bq, wk, bk, wv, bv,
           wo, bo, f1w, f1b, f2w, f2b):
    x4 = x.reshape(T, 4, D)
    vspec = lambda shape: pl.BlockSpec(shape, lambda: tuple(0 for _ in shape))
    any_spec = pl.BlockSpec(memory_space=pl.ANY)
    out = pl.pallas_call(
        _mega_body,
        grid=(),
        compiler_params=pltpu.CompilerParams(vmem_limit_bytes=67108864),
        in_specs=[
            any_spec,
            vspec((D, 1)),
            vspec((D, E)),
            any_spec,
            vspec((E, H)),
            any_spec,
            vspec((E, DOUT)),
            any_spec,
            vspec((1, DOUT)),
            any_spec,
            vspec((1, DOUT)),
            any_spec,
            vspec((1, DOUT)),
            any_spec,
            vspec((1, DOUT)),
            any_spec,
            vspec((1, FFN)),
            any_spec,
            vspec((1, DOUT)),
        ],
        out_specs=pl.BlockSpec((T, DOUT), lambda: (0, 0)),
        out_shape=jax.ShapeDtypeStruct((T, DOUT), jnp.float32),
        scratch_shapes=[
            pltpu.VMEM((2, XTB, 4, D), jnp.float32),
            pltpu.VMEM((T, D), jnp.float32),
            pltpu.VMEM((T, E), jnp.float32),
            pltpu.VMEM((T, DOUT), jnp.float32),
            pltpu.VMEM((T, DOUT), jnp.float32),
            pltpu.VMEM((T, DOUT), jnp.float32),
            pltpu.VMEM((T, DOUT), jnp.float32),
            pltpu.VMEM((NSLOT, 1024, 1024), jnp.float32),
            pltpu.SemaphoreType.DMA((2,)),
            pltpu.SemaphoreType.DMA((NSLOT,)),
        ],
    )(x4, attn_w, gate_w, W1, b1, W2, b2,
      wq, bq.reshape(1, DOUT), wk, bk.reshape(1, DOUT),
      wv, bv.reshape(1, DOUT), wo, bo.reshape(1, DOUT),
      f1w, f1b.reshape(1, FFN), f2w, f2b.reshape(1, DOUT))
    return out.reshape(B, N, DOUT)
